# Initial kernel scaffold; baseline (speedup 1.0000x reference)
import jax
import jax.numpy as jnp
from jax.experimental import pallas as pl


def kernel(x, edge_index, rule_ids, node_labels, weights, bias):
    raise NotImplementedError("write your pallas kernel here")


# speedup vs baseline: 6.6229x; 6.6229x over previous
import functools

import jax
import jax.numpy as jnp
from jax import lax
from jax.experimental import pallas as pl
from jax.experimental.pallas import tpu as pltpu
from jax.experimental.pallas import tpu_sc as plsc

N = 10000
E = 320000
D = 128
N_LABELS = 50
N_RULES = 2500

NC = 2
NS = 16
NW = NC * NS
EPW = E // NW
EPWP = 10240
EP = EPWP * NW
CH = 128
NCHUNK = EPWP // CH
NP = 10240
ROWS_PER_TILE = NP // NS
STRIPE = 128
W_PAD = 2512


def _sc_body(x_hbm, src_hbm, dst_hbm, rule_hbm, w_hbm, out_hbm,
             srcb, dstb, ruleb, w_v, wb16, rows, acc_sh, sem):
    c = lax.axis_index("c")
    s = lax.axis_index("s")
    wid = c * NS + s

    pltpu.sync_copy(w_hbm, w_v)

    zero16 = jnp.zeros((16,), jnp.float32)

    def _zrow(i, carry):
        for cg in range(8):
            rows[i, pl.ds(cg * 16, 16)] = zero16
        return carry

    lax.fori_loop(0, STRIPE, _zrow, 0)
    row0 = s * ROWS_PER_TILE
    for b in range(ROWS_PER_TILE // STRIPE):
        pltpu.sync_copy(rows, acc_sh.at[pl.ds(row0 + b * STRIPE, STRIPE)])
    plsc.subcore_barrier()

    ebase = wid * EPWP

    def _chunk(g, carry):
        off = pl.multiple_of(ebase + g * CH, 128)
        pltpu.sync_copy(src_hbm.at[pl.ds(off, CH)], srcb)
        pltpu.sync_copy(dst_hbm.at[pl.ds(off, CH)], dstb)
        pltpu.sync_copy(rule_hbm.at[pl.ds(off, CH)], ruleb)
        pltpu.async_copy(x_hbm.at[srcb], rows, sem).wait()
        lane = lax.iota(jnp.int32, 16)
        for b in range(CH // 16):
            r16 = ruleb[pl.ds(b * 16, 16)]
            wv = plsc.load_gather(w_v, [r16])
            for rr in range(16):
                r = b * 16 + rr
                wsc = jnp.sum(jnp.where(lane == rr, wv, jnp.float32(0.0)))
                ws = jnp.zeros((16,), jnp.float32) + wsc
                for cg in range(8):
                    rows[r, pl.ds(cg * 16, 16)] = (
                        rows[r, pl.ds(cg * 16, 16)] * ws)
        pltpu.sync_copy(rows, acc_sh.at[dstb], add=True)
        return carry

    lax.fori_loop(0, NCHUNK, _chunk, 0)
    plsc.subcore_barrier()

    for b in range(ROWS_PER_TILE // STRIPE):
        rstart = row0 + b * STRIPE
        pltpu.sync_copy(acc_sh.at[pl.ds(rstart, STRIPE)], rows)
        pltpu.sync_copy(rows, out_hbm.at[c, pl.ds(rstart, STRIPE)])


@jax.jit
def _sc_aggregate(x, src_f, dst_f, rule_f, w_pad):
    mesh = plsc.VectorSubcoreMesh(core_axis_name="c", subcore_axis_name="s")
    f = functools.partial(
        pl.kernel,
        mesh=mesh,
        out_type=jax.ShapeDtypeStruct((NC, NP, D), jnp.float32),
        scratch_types=[
            pltpu.VMEM((CH,), jnp.int32),
            pltpu.VMEM((CH,), jnp.int32),
            pltpu.VMEM((CH,), jnp.int32),
            pltpu.VMEM((W_PAD,), jnp.float32),
            pltpu.VMEM((16,), jnp.float32),
            pltpu.VMEM((CH, D), jnp.float32),
            pltpu.VMEM_SHARED((NP, D), jnp.float32),
            pltpu.SemaphoreType.DMA,
        ],
        compiler_params=pltpu.CompilerParams(needs_layout_passes=False),
    )(_sc_body)
    return f(x, src_f, dst_f, rule_f, w_pad)


def _tc_combine_body(p_ref, lab_ref, bias_ref, o_ref):
    lab = lab_ref[...]
    iota = lax.broadcasted_iota(jnp.int32, (lab.shape[0], D), 1)
    onehot = (lab == iota).astype(jnp.float32)
    bvals = jnp.sum(onehot * bias_ref[...], axis=1, keepdims=True)
    o_ref[...] = p_ref[0] + p_ref[1] + bvals


@jax.jit
def _tc_combine(partials, labels2d, bias_row):
    blk = 1000
    grid = N // blk
    return pl.pallas_call(
        _tc_combine_body,
        grid=(grid,),
        in_specs=[
            pl.BlockSpec((NC, blk, D), lambda i: (0, i, 0)),
            pl.BlockSpec((blk, 1), lambda i: (i, 0)),
            pl.BlockSpec((1, D), lambda i: (0, 0)),
        ],
        out_specs=pl.BlockSpec((blk, D), lambda i: (i, 0)),
        out_shape=jax.ShapeDtypeStruct((N, D), jnp.float32),
    )(partials, labels2d, bias_row)


def kernel(x, edge_index, rule_ids, node_labels, weights, bias):
    pad = EPWP - EPW
    src_f = jnp.pad(edge_index[0].reshape(NW, EPW), ((0, 0), (0, pad))).reshape(EP)
    dst_f = jnp.pad(edge_index[1].reshape(NW, EPW), ((0, 0), (0, pad))).reshape(EP)
    rule_f = jnp.pad(rule_ids.reshape(NW, EPW), ((0, 0), (0, pad)),
                     constant_values=N_RULES).reshape(EP)
    w_pad = jnp.zeros((W_PAD,), jnp.float32).at[:N_RULES].set(weights)
    partials = _sc_aggregate(x, src_f, dst_f, rule_f, w_pad)
    labels2d = node_labels.reshape(N, 1)
    bias_row = jnp.zeros((1, D), jnp.float32).at[0, :N_LABELS].set(bias)
    return _tc_combine(partials, labels2d, bias_row)


# speedup vs baseline: 7.3391x; 1.1081x over previous
import functools

import jax
import jax.numpy as jnp
from jax import lax
from jax.experimental import pallas as pl
from jax.experimental.pallas import tpu as pltpu
from jax.experimental.pallas import tpu_sc as plsc

N = 10000
E = 320000
D = 128
N_LABELS = 50
N_RULES = 2500

NC = 2
NS = 16
NW = NC * NS
EPW = E // NW
EPWP = 10240
EP = EPWP * NW
CH = 128
NCHUNK = EPWP // CH
NP = 10240
ROWS_PER_TILE = NP // NS
STRIPE = 128
W_PAD = 2512


def _sc_body(x_hbm, meta_hbm, w_hbm, out_hbm,
             metab, w_v, rows, acc_sh, sem):
    c = lax.axis_index("c")
    s = lax.axis_index("s")
    wid = c * NS + s

    pltpu.sync_copy(w_hbm, w_v)

    zero16 = jnp.zeros((16,), jnp.float32)

    def _zrow(i, carry):
        for cg in range(8):
            rows[i, pl.ds(cg * 16, 16)] = zero16
        return carry

    lax.fori_loop(0, STRIPE, _zrow, 0)
    row0 = s * ROWS_PER_TILE
    for b in range(ROWS_PER_TILE // STRIPE):
        pltpu.sync_copy(rows, acc_sh.at[pl.ds(row0 + b * STRIPE, STRIPE)])
    plsc.subcore_barrier()

    kbase = wid * NCHUNK

    def _chunk(g, carry):
        pltpu.sync_copy(meta_hbm.at[kbase + g], metab)
        pltpu.async_copy(x_hbm.at[metab.at[0]], rows, sem).wait()
        for b in range(CH // 16):
            r16 = metab[2, pl.ds(b * 16, 16)]
            wv = plsc.load_gather(w_v, [r16])
            for rr in range(16):
                r = b * 16 + rr
                ws = jnp.take_along_axis(
                    wv, jnp.full((16,), rr, jnp.int32), axis=0)
                for cg in range(8):
                    rows[r, pl.ds(cg * 16, 16)] = (
                        rows[r, pl.ds(cg * 16, 16)] * ws)
        pltpu.sync_copy(rows, acc_sh.at[metab.at[1]], add=True)
        return carry

    lax.fori_loop(0, NCHUNK, _chunk, 0)
    plsc.subcore_barrier()

    for b in range(ROWS_PER_TILE // STRIPE):
        rstart = row0 + b * STRIPE
        pltpu.sync_copy(acc_sh.at[pl.ds(rstart, STRIPE)], rows)
        pltpu.sync_copy(rows, out_hbm.at[c, pl.ds(rstart, STRIPE)])


@jax.jit
def _sc_aggregate(x, meta, w_pad):
    mesh = plsc.VectorSubcoreMesh(core_axis_name="c", subcore_axis_name="s")
    f = functools.partial(
        pl.kernel,
        mesh=mesh,
        out_type=jax.ShapeDtypeStruct((NC, NP, D), jnp.float32),
        scratch_types=[
            pltpu.VMEM((3, CH), jnp.int32),
            pltpu.VMEM((W_PAD,), jnp.float32),
            pltpu.VMEM((CH, D), jnp.float32),
            pltpu.VMEM_SHARED((NP, D), jnp.float32),
            pltpu.SemaphoreType.DMA,
        ],
        compiler_params=pltpu.CompilerParams(needs_layout_passes=False),
    )(_sc_body)
    return f(x, meta, w_pad)


def _tc_combine_body(p_ref, lab_ref, bias_ref, o_ref):
    lab = lab_ref[...]
    iota = lax.broadcasted_iota(jnp.int32, (lab.shape[0], D), 1)
    onehot = (lab == iota).astype(jnp.float32)
    bvals = jnp.sum(onehot * bias_ref[...], axis=1, keepdims=True)
    o_ref[...] = p_ref[0] + p_ref[1] + bvals


@jax.jit
def _tc_combine(partials, labels2d, bias_row):
    blk = 1000
    grid = N // blk
    return pl.pallas_call(
        _tc_combine_body,
        grid=(grid,),
        in_specs=[
            pl.BlockSpec((NC, blk, D), lambda i: (0, i, 0)),
            pl.BlockSpec((blk, 1), lambda i: (i, 0)),
            pl.BlockSpec((1, D), lambda i: (0, 0)),
        ],
        out_specs=pl.BlockSpec((blk, D), lambda i: (i, 0)),
        out_shape=jax.ShapeDtypeStruct((N, D), jnp.float32),
    )(partials, labels2d, bias_row)


def kernel(x, edge_index, rule_ids, node_labels, weights, bias):
    pad = EPWP - EPW
    src_f = jnp.pad(edge_index[0].reshape(NW, EPW), ((0, 0), (0, pad)))
    dst_f = jnp.pad(edge_index[1].reshape(NW, EPW), ((0, 0), (0, pad)))
    rule_f = jnp.pad(rule_ids.reshape(NW, EPW), ((0, 0), (0, pad)),
                     constant_values=N_RULES)
    meta = jnp.stack([src_f.reshape(NW * NCHUNK, CH),
                      dst_f.reshape(NW * NCHUNK, CH),
                      rule_f.reshape(NW * NCHUNK, CH)], axis=1)
    w_pad = jnp.zeros((W_PAD,), jnp.float32).at[:N_RULES].set(weights)
    partials = _sc_aggregate(x, meta, w_pad)
    labels2d = node_labels.reshape(N, 1)
    bias_row = jnp.zeros((1, D), jnp.float32).at[0, :N_LABELS].set(bias)
    return _tc_combine(partials, labels2d, bias_row)


# speedup vs baseline: 8.2730x; 1.1272x over previous
import functools

import jax
import jax.numpy as jnp
from jax import lax
from jax.experimental import pallas as pl
from jax.experimental.pallas import tpu as pltpu
from jax.experimental.pallas import tpu_sc as plsc

N = 10000
E = 320000
D = 128
N_LABELS = 50
N_RULES = 2500

NC = 2
NS = 16
NW = NC * NS
EPW = E // NW
EPWP = 10240
EP = EPWP * NW
CH = 128
NCHUNK = EPWP // CH
NP = 10240
ROWS_PER_TILE = NP // NS
STRIPE = 128
W_PAD = 2512


NPAIR = NCHUNK // 2


def _sc_body(x_hbm, meta_hbm, w_hbm, out_hbm,
             metab, w_v, rows_a, rows_b, dst_a, dst_b, acc_sh,
             gsem_a, gsem_b, ssem_a, ssem_b, msem):
    c = lax.axis_index("c")
    s = lax.axis_index("s")
    wid = c * NS + s

    pltpu.sync_copy(w_hbm, w_v)

    zero16 = jnp.zeros((16,), jnp.float32)

    def _zrow(i, carry):
        for cg in range(8):
            rows_a[i, pl.ds(cg * 16, 16)] = zero16
        return carry

    lax.fori_loop(0, STRIPE, _zrow, 0)
    row0 = s * ROWS_PER_TILE
    for b in range(ROWS_PER_TILE // STRIPE):
        pltpu.sync_copy(rows_a, acc_sh.at[pl.ds(row0 + b * STRIPE, STRIPE)])
    plsc.subcore_barrier()

    kbase = wid * NPAIR

    def _scale(rows, slot, j):
        lane = lax.iota(jnp.int32, 16)
        c_row = jnp.full((16,), slot * 6 + j * 3 + 2, jnp.int32)

        def _grp(b16, carry):
            r16 = plsc.load_gather(metab, [c_row, lane + b16 * 16])
            wv = plsc.load_gather(w_v, [r16])
            for rr in range(16):
                r = b16 * 16 + rr
                ws = jnp.take_along_axis(
                    wv, jnp.full((16,), rr, jnp.int32), axis=0)
                for cg in range(8):
                    rows[r, pl.ds(cg * 16, 16)] = (
                        rows[r, pl.ds(cg * 16, 16)] * ws)
            return carry

        lax.fori_loop(0, CH // 16, _grp, 0)

    def _copy_dst(dstb, slot, j):
        row = slot * 6 + j * 3 + 1
        for i in range(CH // 16):
            dstb[pl.ds(i * 16, 16)] = metab[row, pl.ds(i * 16, 16)]

    pltpu.sync_copy(meta_hbm.at[kbase], metab.at[pl.ds(0, 6)])
    pltpu.async_copy(x_hbm.at[metab.at[0]], rows_a, gsem_a)

    def _pair(p, a):
        b = 1 - a
        last = p == NPAIR - 1

        @pl.when(jnp.logical_not(last))
        def _():
            pltpu.async_copy(meta_hbm.at[kbase + p + 1],
                             metab.at[pl.ds(b * 6, 6)], msem)

        pltpu.make_async_copy(x_hbm.at[metab.at[a * 6]], rows_a,
                              gsem_a).wait()
        _copy_dst(dst_a, a, 0)
        _scale(rows_a, a, 0)

        @pl.when(p > 0)
        def _():
            pltpu.make_async_copy(rows_b, acc_sh.at[dst_b], ssem_b).wait()

        pltpu.async_copy(x_hbm.at[metab.at[a * 6 + 3]], rows_b, gsem_b)
        pltpu.async_copy(rows_a, acc_sh.at[dst_a], ssem_a, add=True)

        pltpu.make_async_copy(x_hbm.at[metab.at[a * 6 + 3]], rows_b,
                              gsem_b).wait()
        _copy_dst(dst_b, a, 1)
        _scale(rows_b, a, 1)

        pltpu.make_async_copy(rows_a, acc_sh.at[dst_a], ssem_a).wait()

        @pl.when(jnp.logical_not(last))
        def _():
            pltpu.make_async_copy(meta_hbm.at[kbase + p + 1],
                                  metab.at[pl.ds(b * 6, 6)], msem).wait()
            pltpu.async_copy(x_hbm.at[metab.at[b * 6]], rows_a, gsem_a)

        pltpu.async_copy(rows_b, acc_sh.at[dst_b], ssem_b, add=True)

    def _quad(q, carry):
        _pair(2 * q, 0)
        _pair(2 * q + 1, 1)
        return carry

    lax.fori_loop(0, NPAIR // 2, _quad, 0)
    pltpu.make_async_copy(rows_b, acc_sh.at[dst_b], ssem_b).wait()
    plsc.subcore_barrier()

    for b in range(ROWS_PER_TILE // STRIPE):
        rstart = row0 + b * STRIPE
        pltpu.sync_copy(acc_sh.at[pl.ds(rstart, STRIPE)], rows_a)
        pltpu.sync_copy(rows_a, out_hbm.at[c, pl.ds(rstart, STRIPE)])


@jax.jit
def _sc_aggregate(x, meta, w_pad):
    mesh = plsc.VectorSubcoreMesh(core_axis_name="c", subcore_axis_name="s")
    f = functools.partial(
        pl.kernel,
        mesh=mesh,
        out_type=jax.ShapeDtypeStruct((NC, NP, D), jnp.float32),
        scratch_types=[
            pltpu.VMEM((12, CH), jnp.int32),
            pltpu.VMEM((W_PAD,), jnp.float32),
            pltpu.VMEM((CH, D), jnp.float32),
            pltpu.VMEM((CH, D), jnp.float32),
            pltpu.VMEM((CH,), jnp.int32),
            pltpu.VMEM((CH,), jnp.int32),
            pltpu.VMEM_SHARED((NP, D), jnp.float32),
            pltpu.SemaphoreType.DMA,
            pltpu.SemaphoreType.DMA,
            pltpu.SemaphoreType.DMA,
            pltpu.SemaphoreType.DMA,
            pltpu.SemaphoreType.DMA,
        ],
        compiler_params=pltpu.CompilerParams(needs_layout_passes=False),
    )(_sc_body)
    return f(x, meta, w_pad)


def _tc_combine_body(p_ref, lab_ref, bias_ref, o_ref):
    lab = lab_ref[...]
    iota = lax.broadcasted_iota(jnp.int32, (lab.shape[0], D), 1)
    onehot = (lab == iota).astype(jnp.float32)
    bvals = jnp.sum(onehot * bias_ref[...], axis=1, keepdims=True)
    o_ref[...] = p_ref[0] + p_ref[1] + bvals


@jax.jit
def _tc_combine(partials, labels2d, bias_row):
    blk = 1000
    grid = N // blk
    return pl.pallas_call(
        _tc_combine_body,
        grid=(grid,),
        in_specs=[
            pl.BlockSpec((NC, blk, D), lambda i: (0, i, 0)),
            pl.BlockSpec((blk, 1), lambda i: (i, 0)),
            pl.BlockSpec((1, D), lambda i: (0, 0)),
        ],
        out_specs=pl.BlockSpec((blk, D), lambda i: (i, 0)),
        out_shape=jax.ShapeDtypeStruct((N, D), jnp.float32),
    )(partials, labels2d, bias_row)


def kernel(x, edge_index, rule_ids, node_labels, weights, bias):
    pad = EPWP - EPW
    src_f = jnp.pad(edge_index[0].reshape(NW, EPW), ((0, 0), (0, pad)))
    dst_f = jnp.pad(edge_index[1].reshape(NW, EPW), ((0, 0), (0, pad)))
    rule_f = jnp.pad(rule_ids.reshape(NW, EPW), ((0, 0), (0, pad)),
                     constant_values=N_RULES)
    meta = jnp.stack([src_f.reshape(NW * NCHUNK, CH),
                      dst_f.reshape(NW * NCHUNK, CH),
                      rule_f.reshape(NW * NCHUNK, CH)],
                     axis=1).reshape(NW * NPAIR, 6, CH)
    w_pad = jnp.zeros((W_PAD,), jnp.float32).at[:N_RULES].set(weights)
    partials = _sc_aggregate(x, meta, w_pad)
    labels2d = node_labels.reshape(N, 1)
    bias_row = jnp.zeros((1, D), jnp.float32).at[0, :N_LABELS].set(bias)
    return _tc_combine(partials, labels2d, bias_row)
